# Initial kernel scaffold; baseline (speedup 1.0000x reference)
import jax
import jax.numpy as jnp
from jax.experimental import pallas as pl


def kernel(QK, pos, table):
    raise NotImplementedError("write your pallas kernel here")


# speedup vs baseline: 80.3966x; 80.3966x over previous
import functools

import jax
import jax.numpy as jnp
from jax.experimental import pallas as pl

N, H, L, S = 1, 12, 2048, 2048
MAX_SPATIAL = 100
BL = 256


def _body(tab_ref, pos_ref, qk_ref, out_ref):
    row = tab_ref[0]
    bc = jnp.broadcast_to(row, (BL, 128))
    for c in range(S // 128):
        sl = pl.ds(c * 128, 128)
        idx = pos_ref[0, :, sl]
        emb = jnp.take_along_axis(bc, idx, axis=1)
        out_ref[0, 0, :, sl] = qk_ref[0, 0, :, sl] + emb


@jax.jit
def kernel(QK, pos, table):
    tabT = jnp.zeros((H, 1, 128), dtype=table.dtype).at[:, 0, :MAX_SPATIAL].set(table.T)
    grid = (L // BL, H)
    out = pl.pallas_call(
        _body,
        grid=grid,
        in_specs=[
            pl.BlockSpec((1, 1, 128), lambda b, h: (h, 0, 0)),
            pl.BlockSpec((1, BL, S), lambda b, h: (0, b, 0)),
            pl.BlockSpec((1, 1, BL, S), lambda b, h: (0, h, b, 0)),
        ],
        out_specs=pl.BlockSpec((1, 1, BL, S), lambda b, h: (0, h, b, 0)),
        out_shape=jax.ShapeDtypeStruct((N, H, L, S), QK.dtype),
    )(tabT, pos, QK)
    return out
